# baseline (speedup 1.0000x reference)
import functools

import jax
import jax.numpy as jnp
from jax import lax
from jax.experimental import pallas as pl
from jax.experimental.pallas import tpu as pltpu
from jax.experimental.pallas import tpu_sc as plsc

_LANES = 16


_BK = 128


def _mm_body(a_ref, b_hbm, o_ref, b_res, st0, st1, sem0, sem1):
    ni = pl.program_id(0)
    mi = pl.program_id(1)
    bn = b_res.shape[1]
    k = b_res.shape[0]
    a_bf = a_ref[...].astype(jnp.bfloat16)

    @pl.when(mi == 0)
    def _first():
        nstage = k // _BK
        sts = (st0, st1)
        sems = (sem0, sem1)

        def _cp(p):
            return pltpu.make_async_copy(
                b_hbm.at[pl.ds(p * _BK, _BK), pl.ds(ni * bn, bn)],
                sts[p % 2], sems[p % 2])

        cps = {0: _cp(0)}
        cps[0].start()
        acc = None
        for p in range(nstage):
            if p + 1 < nstage:
                cps[p + 1] = _cp(p + 1)
                cps[p + 1].start()
            cps.pop(p).wait()
            panel = sts[p % 2][...].astype(jnp.bfloat16)
            b_res[pl.ds(p * _BK, _BK), :] = panel
            d = jnp.dot(a_bf[:, p * _BK:(p + 1) * _BK], panel,
                        preferred_element_type=jnp.float32)
            acc = d if acc is None else acc + d
        o_ref[...] = pltpu.bitcast(acc.astype(jnp.bfloat16), jnp.int32)

    @pl.when(mi != 0)
    def _rest():
        acc = None
        for p in range(k // 512):
            d = jnp.dot(a_bf[:, p * 512:(p + 1) * 512],
                        b_res[p * 512:(p + 1) * 512, :],
                        preferred_element_type=jnp.float32)
            acc = d if acc is None else acc + d
        o_ref[...] = pltpu.bitcast(acc.astype(jnp.bfloat16), jnp.int32)


def _matmul(a, b, m_out, bm=256, bn=2048):
    k = a.shape[1]
    n = b.shape[1]
    return pl.pallas_call(
        _mm_body,
        grid=(n // bn, m_out // bm),
        in_specs=[
            pl.BlockSpec((bm, k), lambda ni, mi: (mi, 0)),
            pl.BlockSpec(memory_space=pltpu.MemorySpace.HBM),
        ],
        out_specs=pl.BlockSpec((bm // 2, bn), lambda ni, mi: (mi, ni)),
        out_shape=jax.ShapeDtypeStruct((m_out // 2, n), jnp.int32),
        scratch_shapes=[
            pltpu.VMEM((k, bn), jnp.bfloat16),
            pltpu.VMEM((_BK, bn), jnp.float32),
            pltpu.VMEM((_BK, bn), jnp.float32),
            pltpu.SemaphoreType.DMA,
            pltpu.SemaphoreType.DMA,
        ],
    )(a, b)


_NW = 32
_RB = 16


def _gather_body(nwords, ncand, rpw, score_hbm, items_hbm, out_hbm,
                 items_v, buf0, buf1, out_v, sem0, sem1):
    cid = lax.axis_index("c")
    sid = lax.axis_index("s")
    wid = sid * 2 + cid
    row0 = wid * rpw
    ncol = ncand + 1
    nchunk = (ncand + _LANES - 1) // _LANES

    pltpu.sync_copy(items_hbm.at[pl.ds(row0, rpw)], items_v)

    bufs = (buf0, buf1)
    sems = (sem0, sem1)
    nblk = rpw // _RB
    prow0 = wid * (rpw // 2)
    prb = _RB // 2
    handles = {0: pltpu.async_copy(score_hbm.at[pl.ds(prow0, prb)], buf0, sem0)}
    for blk in range(nblk):
        if blk + 1 < nblk:
            handles[blk + 1] = pltpu.async_copy(
                score_hbm.at[pl.ds(prow0 + (blk + 1) * prb, prb)],
                bufs[(blk + 1) % 2], sems[(blk + 1) % 2])
        handles.pop(blk).wait()
        buf = bufs[blk % 2]

        def row_body(rr, carry, blk=blk, buf=buf):
            rowc = blk * _RB + rr
            odd = lax.bitwise_and(rr, 1) == 1
            rsplat = (jnp.full((_LANES,), 0, jnp.int32)
                      + lax.shift_right_logical(rr, 1))
            csplat = jnp.full((_LANES,), 0, jnp.int32) + rowc
            lanes = lax.iota(jnp.int32, _LANES)
            for c in range(nchunk):
                j = c * _LANES + lanes
                col = jnp.minimum(1 + j, ncol - 1)
                cand = plsc.load_gather(items_v, [csplat, col])
                word = plsc.load_gather(buf, [rsplat, cand])
                hi = jnp.where(odd, word, lax.shift_left(word, 16))
                vals = plsc.bitcast(
                    lax.bitwise_and(hi, jnp.int32(-65536)), jnp.float32)
                if (c + 1) * _LANES <= ncand:
                    plsc.store_scatter(out_v, [csplat, j], vals)
                else:
                    plsc.store_scatter(out_v, [csplat, j], vals,
                                       mask=j < ncand)
            return carry

        lax.fori_loop(0, _RB, row_body, 0)

    pltpu.sync_copy(out_v, out_hbm.at[pl.ds(row0, rpw)])


def _gather(score, test_sample):
    b, ncol = test_sample.shape
    ncand = ncol - 1
    nwords = score.shape[1]
    rpw = b // _NW
    mesh = plsc.VectorSubcoreMesh(core_axis_name="c", subcore_axis_name="s")
    f = pl.kernel(
        functools.partial(_gather_body, nwords, ncand, rpw),
        out_type=jax.ShapeDtypeStruct((b, ncand), jnp.float32),
        mesh=mesh,
        compiler_params=pltpu.CompilerParams(needs_layout_passes=False),
        scratch_types=[
            pltpu.VMEM((rpw, ncol), jnp.int32),
            pltpu.VMEM((_RB // 2, nwords), jnp.int32),
            pltpu.VMEM((_RB // 2, nwords), jnp.int32),
            pltpu.VMEM((rpw, ncand), jnp.float32),
            pltpu.SemaphoreType.DMA,
            pltpu.SemaphoreType.DMA,
        ],
    )
    return f(score, test_sample)


def kernel(mat, sim, test_sample):
    b = test_sample.shape[0]
    score = _matmul(mat, sim, b)
    return _gather(score, test_sample)


# speedup vs baseline: 1.1300x; 1.1300x over previous
import functools

import jax
import jax.numpy as jnp
from jax import lax
from jax.experimental import pallas as pl
from jax.experimental.pallas import tpu as pltpu
from jax.experimental.pallas import tpu_sc as plsc

_LANES = 16


_BK = 256


def _mm_body(a_ref, b_hbm, o_ref, b_res, st0, st1, sem0, sem1):
    mi = pl.program_id(0)
    a_bf = a_ref[...].astype(jnp.bfloat16)

    @pl.when(mi == 0)
    def _first():
        k = b_res.shape[0]
        nstage = k // _BK
        sts = (st0, st1)
        sems = (sem0, sem1)
        cps = {0: pltpu.make_async_copy(b_hbm.at[pl.ds(0, _BK)], st0, sem0)}
        cps[0].start()
        acc = None
        for p in range(nstage):
            if p + 1 < nstage:
                cp = pltpu.make_async_copy(
                    b_hbm.at[pl.ds((p + 1) * _BK, _BK)],
                    sts[(p + 1) % 2], sems[(p + 1) % 2])
                cp.start()
                cps[p + 1] = cp
            cps.pop(p).wait()
            panel = sts[p % 2][...].astype(jnp.bfloat16)
            b_res[pl.ds(p * _BK, _BK), :] = panel
            d = jnp.dot(a_bf[:, p * _BK:(p + 1) * _BK], panel,
                        preferred_element_type=jnp.float32)
            acc = d if acc is None else acc + d
        o_ref[...] = pltpu.bitcast(acc.astype(jnp.bfloat16), jnp.int32)

    @pl.when(mi != 0)
    def _rest():
        acc = jnp.dot(a_bf, b_res[...], preferred_element_type=jnp.float32)
        o_ref[...] = pltpu.bitcast(acc.astype(jnp.bfloat16), jnp.int32)


def _matmul(a, b, m_out, bm=128):
    k = a.shape[1]
    n = b.shape[1]
    return pl.pallas_call(
        _mm_body,
        grid=(m_out // bm,),
        in_specs=[
            pl.BlockSpec((bm, k), lambda mi: (mi, 0)),
            pl.BlockSpec(memory_space=pltpu.MemorySpace.HBM),
        ],
        out_specs=pl.BlockSpec((bm // 2, n), lambda mi: (mi, 0)),
        out_shape=jax.ShapeDtypeStruct((m_out // 2, n), jnp.int32),
        scratch_shapes=[
            pltpu.VMEM((k, n), jnp.bfloat16),
            pltpu.VMEM((_BK, n), jnp.float32),
            pltpu.VMEM((_BK, n), jnp.float32),
            pltpu.SemaphoreType.DMA,
            pltpu.SemaphoreType.DMA,
        ],
    )(a, b)


_NW = 32
_RB = 16


def _gather_body(nwords, ncand, rpw, score_hbm, items_hbm, out_hbm,
                 items_v, buf0, buf1, out_v, sem0, sem1):
    cid = lax.axis_index("c")
    sid = lax.axis_index("s")
    wid = sid * 2 + cid
    row0 = wid * rpw
    ncol = ncand + 1
    nchunk = (ncand + _LANES - 1) // _LANES

    bufs = (buf0, buf1)
    sems = (sem0, sem1)
    nblk = rpw // _RB
    prow0 = wid * (rpw // 2)
    prb = _RB // 2
    handles = {0: pltpu.async_copy(score_hbm.at[pl.ds(prow0, prb)], buf0, sem0)}
    pltpu.sync_copy(items_hbm.at[pl.ds(row0, rpw)], items_v)
    for blk in range(nblk):
        if blk + 1 < nblk:
            handles[blk + 1] = pltpu.async_copy(
                score_hbm.at[pl.ds(prow0 + (blk + 1) * prb, prb)],
                bufs[(blk + 1) % 2], sems[(blk + 1) % 2])
        handles.pop(blk).wait()
        buf = bufs[blk % 2]

        def row_body(rr, carry, blk=blk, buf=buf):
            rowc = blk * _RB + rr
            odd = lax.bitwise_and(rr, 1) == 1
            rsplat = (jnp.full((_LANES,), 0, jnp.int32)
                      + lax.shift_right_logical(rr, 1))
            csplat = jnp.full((_LANES,), 0, jnp.int32) + rowc
            lanes = lax.iota(jnp.int32, _LANES)
            for c in range(nchunk):
                j = c * _LANES + lanes
                col = jnp.minimum(1 + j, ncol - 1)
                cand = plsc.load_gather(items_v, [csplat, col])
                word = plsc.load_gather(buf, [rsplat, cand])
                hi = jnp.where(odd, word, lax.shift_left(word, 16))
                vals = plsc.bitcast(
                    lax.bitwise_and(hi, jnp.int32(-65536)), jnp.float32)
                if (c + 1) * _LANES <= ncand:
                    plsc.store_scatter(out_v, [csplat, j], vals)
                else:
                    plsc.store_scatter(out_v, [csplat, j], vals,
                                       mask=j < ncand)
            return carry

        lax.fori_loop(0, _RB, row_body, 0)

    pltpu.sync_copy(out_v, out_hbm.at[pl.ds(row0, rpw)])


def _gather(score, test_sample):
    b, ncol = test_sample.shape
    ncand = ncol - 1
    nwords = score.shape[1]
    rpw = b // _NW
    mesh = plsc.VectorSubcoreMesh(core_axis_name="c", subcore_axis_name="s")
    f = pl.kernel(
        functools.partial(_gather_body, nwords, ncand, rpw),
        out_type=jax.ShapeDtypeStruct((b, ncand), jnp.float32),
        mesh=mesh,
        compiler_params=pltpu.CompilerParams(needs_layout_passes=False),
        scratch_types=[
            pltpu.VMEM((rpw, ncol), jnp.int32),
            pltpu.VMEM((_RB // 2, nwords), jnp.int32),
            pltpu.VMEM((_RB // 2, nwords), jnp.int32),
            pltpu.VMEM((rpw, ncand), jnp.float32),
            pltpu.SemaphoreType.DMA,
            pltpu.SemaphoreType.DMA,
        ],
    )
    return f(score, test_sample)


def kernel(mat, sim, test_sample):
    b = test_sample.shape[0]
    score = _matmul(mat, sim, b)
    return _gather(score, test_sample)
